# baseline (speedup 1.0000x reference)
import jax
import jax.numpy as jnp
from jax import lax
from jax.experimental import pallas as pl
from jax.experimental.pallas import tpu as pltpu
from jax.experimental.pallas import tpu_sc as plsc

SEQ = 200
DIM = 64
BATCH = 4096
NUM_CORES = 2
NUM_SUBCORES = 16
NUM_WORKERS = NUM_CORES * NUM_SUBCORES
SEQ_PER_W = BATCH // NUM_WORKERS
G = 2
GROUP_ROWS = G * SEQ
NGROUPS = SEQ_PER_W // G
RING = 4
LANES = 16


def _body(table_hbm, idx_hbm, pos2_hbm, out_hbm,
          idx_bufs, row_bufs, isems, gsems, wsems, fsems):
  wid = lax.axis_index("s") * NUM_CORES + lax.axis_index("c")
  row0 = wid * (SEQ_PER_W * SEQ)

  def idx_start(g, j):
    pltpu.async_copy(
        idx_hbm.at[pl.ds(row0 + g * GROUP_ROWS, GROUP_ROWS)],
        idx_bufs[j], isems[j])

  def idx_wait(j):
    pltpu.make_async_copy(
        idx_hbm.at[pl.ds(row0, GROUP_ROWS)], idx_bufs[j], isems[j]).wait()

  def gather_start(j):
    pltpu.async_copy(table_hbm.at[idx_bufs[j]], row_bufs[j], gsems[j],
                     add=True)

  def gather_wait(j):
    pltpu.make_async_copy(
        table_hbm.at[idx_bufs[j]], row_bufs[j], gsems[j]).wait()

  def wb_start(g, j):
    pltpu.async_copy(
        row_bufs[j],
        out_hbm.at[pl.ds(row0 + g * GROUP_ROWS, GROUP_ROWS)], wsems[j])

  def wb_wait(j):
    pltpu.make_async_copy(
        row_bufs[j], out_hbm.at[pl.ds(row0, GROUP_ROWS)], wsems[j]).wait()

  def fill_start(j):
    pltpu.async_copy(pos2_hbm, row_bufs[j], fsems[j])

  def fill_wait(j):
    pltpu.make_async_copy(pos2_hbm, row_bufs[j], fsems[j]).wait()

  def step(g, j, *, prefetch=True, stage=True, stage_wb_wait=True):
    gather_wait(j)
    wb_start(g, j)
    if prefetch:
      j2 = (j + 2) % RING
      fill_wait(j2)
      idx_wait(j2)
      gather_start(j2)
    if stage:
      j3 = (j + 3) % RING
      if stage_wb_wait:
        wb_wait(j3)
      fill_start(j3)
      idx_start(g + 3, j3)

  for j in range(RING):
    idx_start(j, j)
    fill_start(j)
  for j in range(2):
    idx_wait(j)
    fill_wait(j)
    gather_start(j)

  step(0, 0, stage=False)
  step(1, 1, stage=True)
  step(2, 2, stage=True)
  step(3, 3, stage=True)

  def outer(i, _):
    for j in range(RING):
      step(4 * i + j, j)
    return 0

  lax.fori_loop(1, NGROUPS // RING - 1, outer, 0)

  step(NGROUPS - 4, 0, prefetch=True, stage=True)
  step(NGROUPS - 3, 1, prefetch=True, stage=False)
  step(NGROUPS - 2, 2, prefetch=False, stage=False)
  step(NGROUPS - 1, 3, prefetch=False, stage=False)

  for j in range(RING):
    wb_wait(j)


@jax.jit
def _run(inputs_flat, token_table, pos2):
  mesh = plsc.VectorSubcoreMesh(core_axis_name="c", subcore_axis_name="s")
  kfn = pl.kernel(
      _body,
      out_type=jax.ShapeDtypeStruct((BATCH * SEQ, DIM), jnp.float32),
      mesh=mesh,
      scratch_types=[
          [pltpu.VMEM((GROUP_ROWS,), jnp.int32)] * RING,
          [pltpu.VMEM((GROUP_ROWS, DIM), jnp.float32)] * RING,
          [pltpu.SemaphoreType.DMA] * RING,
          [pltpu.SemaphoreType.DMA] * RING,
          [pltpu.SemaphoreType.DMA] * RING,
          [pltpu.SemaphoreType.DMA] * RING,
      ],
      compiler_params=pltpu.CompilerParams(use_tc_tiling_on_sc=False),
  )
  return kfn(token_table, inputs_flat, pos2)


def kernel(inputs, token_table, pos_table):
  inputs_flat = inputs.reshape(BATCH * SEQ)
  pos2 = jnp.tile(pos_table, (G, 1))
  out = _run(inputs_flat, token_table, pos2)
  return out.reshape(BATCH, SEQ, DIM)


# speedup vs baseline: 1.4499x; 1.4499x over previous
import jax
import jax.numpy as jnp
from jax import lax
from jax.experimental import pallas as pl
from jax.experimental.pallas import tpu as pltpu
from jax.experimental.pallas import tpu_sc as plsc

SEQ = 200
DIM = 64
BATCH = 4096
NUM_CORES = 2
NUM_SUBCORES = 16
NUM_WORKERS = NUM_CORES * NUM_SUBCORES
SEQ_PER_W = BATCH // NUM_WORKERS
G = 2
GROUP_ROWS = G * SEQ
NGROUPS = SEQ_PER_W // G
RING = 4
LANES = 16


def _body(table_hbm, idx_hbm, pos_hbm, out_hbm,
          pos_v, idx_bufs, row_bufs, isems, gsems, wsems):
  wid = lax.axis_index("s") * NUM_CORES + lax.axis_index("c")
  row0 = wid * (SEQ_PER_W * SEQ)

  pltpu.sync_copy(pos_hbm, pos_v)

  def idx_start(g, j):
    pltpu.async_copy(
        idx_hbm.at[pl.ds(row0 + g * GROUP_ROWS, GROUP_ROWS)],
        idx_bufs[j], isems[j])

  def idx_wait(j):
    pltpu.make_async_copy(
        idx_hbm.at[pl.ds(row0, GROUP_ROWS)], idx_bufs[j], isems[j]).wait()

  def gather_start(j):
    pltpu.async_copy(table_hbm.at[idx_bufs[j]], row_bufs[j], gsems[j],
                     add=True)

  def gather_wait(j):
    pltpu.make_async_copy(
        table_hbm.at[idx_bufs[j]], row_bufs[j], gsems[j]).wait()

  def wb_start(g, j):
    pltpu.async_copy(
        row_bufs[j],
        out_hbm.at[pl.ds(row0 + g * GROUP_ROWS, GROUP_ROWS)], wsems[j])

  def wb_wait(j):
    pltpu.make_async_copy(
        row_bufs[j], out_hbm.at[pl.ds(row0, GROUP_ROWS)], wsems[j]).wait()

  def fill_pos(j):
    rows = row_bufs[j]

    def row_fn(r, _):
      for k in range(DIM // LANES):
        sl = pl.ds(k * LANES, LANES)
        p = pos_v[r, sl]
        rows[r, sl] = p
        rows[SEQ + r, sl] = p
      return 0

    lax.fori_loop(0, SEQ, row_fn, 0, unroll=2)

  def step(g, j, *, first_wb_wait, prefetch, stage_idx):
    gather_wait(j)
    wb_start(g, j)
    if prefetch:
      j2 = (j + 2) % RING
      if first_wb_wait:
        wb_wait(j2)
      idx_wait(j2)
      fill_pos(j2)
      gather_start(j2)
    if stage_idx:
      idx_start(g + RING, j)

  for j in range(RING):
    idx_start(j, j)
  for j in range(2):
    idx_wait(j)
    fill_pos(j)
    gather_start(j)

  step(0, 0, first_wb_wait=False, prefetch=True, stage_idx=True)
  step(1, 1, first_wb_wait=False, prefetch=True, stage_idx=True)
  step(2, 2, first_wb_wait=True, prefetch=True, stage_idx=True)
  step(3, 3, first_wb_wait=True, prefetch=True, stage_idx=True)

  def outer(i, _):
    for j in range(RING):
      step(4 * i + j, j, first_wb_wait=True, prefetch=True, stage_idx=True)
    return 0

  lax.fori_loop(1, NGROUPS // RING - 1, outer, 0)

  step(NGROUPS - 4, 0, first_wb_wait=True, prefetch=True, stage_idx=False)
  step(NGROUPS - 3, 1, first_wb_wait=True, prefetch=True, stage_idx=False)
  step(NGROUPS - 2, 2, first_wb_wait=True, prefetch=False, stage_idx=False)
  step(NGROUPS - 1, 3, first_wb_wait=True, prefetch=False, stage_idx=False)

  for j in range(RING):
    wb_wait(j)


@jax.jit
def _run(inputs_flat, token_table, pos_table):
  mesh = plsc.VectorSubcoreMesh(core_axis_name="c", subcore_axis_name="s")
  kfn = pl.kernel(
      _body,
      out_type=jax.ShapeDtypeStruct((BATCH * SEQ, DIM), jnp.float32),
      mesh=mesh,
      scratch_types=[
          pltpu.VMEM((SEQ, DIM), jnp.float32),
          [pltpu.VMEM((GROUP_ROWS,), jnp.int32)] * RING,
          [pltpu.VMEM((GROUP_ROWS, DIM), jnp.float32)] * RING,
          [pltpu.SemaphoreType.DMA] * RING,
          [pltpu.SemaphoreType.DMA] * RING,
          [pltpu.SemaphoreType.DMA] * RING,
      ],
      compiler_params=pltpu.CompilerParams(use_tc_tiling_on_sc=False),
  )
  return kfn(token_table, inputs_flat, pos_table)


def kernel(inputs, token_table, pos_table):
  inputs_flat = inputs.reshape(BATCH * SEQ)
  out = _run(inputs_flat, token_table, pos_table)
  return out.reshape(BATCH, SEQ, DIM)
